# baseline (speedup 1.0000x reference)
import functools

import jax
import jax.numpy as jnp
from jax import lax
from jax.experimental import pallas as pl
from jax.experimental.pallas import tpu as pltpu
from jax.experimental.pallas import tpu_sc as plsc

N = 10000
E = 160000
D = 256
H = 128
N_PAD = 10240
RPT = N_PAD // 16
E_PER_TILE = 10240
E_PAD = E_PER_TILE * 16
BN = 512
NBUF = 4


def _make_sc_aggregate(compute_deg: bool, chunk: int, groups: int):
    gchunks = E_PER_TILE // (groups * chunk)
    assert gchunks % NBUF == 0
    mesh = plsc.VectorSubcoreMesh(core_axis_name="c", subcore_axis_name="s",
                                  num_cores=2, num_subcores=16)
    out_type = [jax.ShapeDtypeStruct((2, N_PAD, H), jnp.float32)]
    scratch = (
        [pltpu.VMEM((gchunks, chunk), jnp.int32)] * 2 +
        [pltpu.VMEM((chunk, H), jnp.float32)] * NBUF +
        [pltpu.VMEM_SHARED((N_PAD, H), jnp.float32)] +
        [pltpu.SemaphoreType.DMA] * (2 * NBUF)
    )
    if compute_deg:
        out_type.append(jax.ShapeDtypeStruct((N_PAD,), jnp.float32))
        out_type.append(jax.ShapeDtypeStruct((16, N_PAD), jnp.float32))
        scratch += [
            pltpu.VMEM((N_PAD,), jnp.float32),
        ]

    def body(x2n, src_idx, dst_idx, *rest):
        if compute_deg:
            agg_out, deg_out, hstage = rest[0], rest[1], rest[2]
            rest = rest[3:]
        else:
            agg_out = rest[0]
            rest = rest[1:]
        src_v, dst_v = rest[0], rest[1]
        bufs = rest[2:2 + NBUF]
        agg_sh = rest[2 + NBUF]
        gsem = rest[3 + NBUF:3 + 2 * NBUF]
        ssem = rest[3 + 2 * NBUF:3 + 3 * NBUF]
        if compute_deg:
            hist_v, = rest[3 + 3 * NBUF:]

        c = lax.axis_index("c")
        s = lax.axis_index("s")
        base = s * RPT
        zeros16 = jnp.zeros((16,), jnp.float32)
        for r in range(16):
            for k in range(H // 16):
                bufs[0][r, pl.ds(k * 16, 16)] = zeros16

        if compute_deg:
            def zero_hist(g, carry):
                hist_v[pl.ds(g * 16, 16)] = zeros16
                return carry
            lax.fori_loop(0, N_PAD // 16, zero_hist, 0)

        def zero_agg(i, carry):
            pltpu.sync_copy(bufs[0].at[pl.ds(0, 16)],
                            agg_sh.at[pl.ds(base + i * 16, 16)])
            return carry
        lax.fori_loop(0, RPT // 16, zero_agg, 0)

        plsc.subcore_barrier()

        def hist_chunk(j):
            if not compute_deg:
                return

            @pl.when(c == 0)
            def _():
                iota16 = lax.iota(jnp.int32, 16)
                one16 = jnp.ones((16,), jnp.float32)
                zero16 = jnp.zeros((16,), jnp.float32)
                for k in range(chunk // 16):
                    dvec = dst_v[j, pl.ds(k * 16, 16)]
                    for l in range(16):
                        idx = dvec[l]
                        wbase = lax.bitwise_and(idx, ~15)
                        lane = lax.bitwise_and(idx, 15)
                        oh = jnp.where(iota16 == lane, one16, zero16)
                        w = hist_v[pl.ds(wbase, 16)]
                        hist_v[pl.ds(wbase, 16)] = w + oh

        def gather(j, q):
            pltpu.async_copy(x2n.at[src_v.at[j]], bufs[q], gsem[q])

        def wait(sem, q):
            pltpu.make_async_copy(x2n.at[pl.ds(0, chunk)], bufs[q],
                                  sem).wait()

        def scatter(j, q):
            pltpu.async_copy(bufs[q], agg_sh.at[dst_v.at[j]], ssem[q],
                             add=True)

        nt = gchunks // NBUF

        def group(g, carry):
            pltpu.sync_copy(src_idx.at[c, s, g], src_v)
            pltpu.sync_copy(dst_idx.at[s, g], dst_v)
            gather(0, 0)
            gather(1, 1)

            def quad(t, carry2):
                j0 = t * NBUF
                for q in (0, 1):
                    wait(gsem[q], q)
                    scatter(j0 + q, q)
                    hist_chunk(j0 + q)

                    @pl.when(t > 0)
                    def _():
                        wait(ssem[q + 2], q + 2)
                    gather(j0 + q + 2, q + 2)
                for q in (2, 3):
                    wait(gsem[q], q)
                    scatter(j0 + q, q)
                    hist_chunk(j0 + q)
                    wait(ssem[q - 2], q - 2)

                    @pl.when(t < nt - 1)
                    def _():
                        gather(j0 + q + 2, q - 2)
                return carry2
            lax.fori_loop(0, nt, quad, 0)
            wait(ssem[2], 2)
            wait(ssem[3], 3)
            return carry
        lax.fori_loop(0, groups, group, 0)

        if compute_deg:
            @pl.when(c == 0)
            def _():
                pltpu.sync_copy(hist_v, hstage.at[s])
        plsc.subcore_barrier()

        if compute_deg:
            @pl.when(c == 0)
            def _():
                def zero_dsum(g, carry):
                    hist_v[pl.ds(g * 16, 16)] = zeros16
                    return carry
                lax.fori_loop(0, RPT // 16, zero_dsum, 0)
                for t in range(16):
                    pltpu.sync_copy(hstage.at[t, pl.ds(base, RPT)],
                                    hist_v.at[pl.ds(RPT, RPT)])

                    def acc(g, carry):
                        hist_v[pl.ds(g * 16, 16)] = (
                            hist_v[pl.ds(g * 16, 16)]
                            + hist_v[pl.ds(RPT + g * 16, 16)])
                        return carry
                    lax.fori_loop(0, RPT // 16, acc, 0)
                pltpu.sync_copy(hist_v.at[pl.ds(0, RPT)],
                                deg_out.at[pl.ds(base, RPT)])

        pltpu.sync_copy(agg_sh.at[pl.ds(base, RPT)],
                        agg_out.at[c, pl.ds(base, RPT)])

    return pl.kernel(body, out_type=out_type, mesh=mesh,
                     scratch_types=scratch)


def _dense_body(paired_out, agg_ref, deg_ref, xr_ref, wl_ref, wr_ref, b_ref,
                out_ref):
    aggf = jnp.concatenate([agg_ref[0], agg_ref[1]], axis=1)
    xf = jnp.concatenate([xr_ref[0], xr_ref[1]], axis=1)
    inv = 1.0 / jnp.maximum(deg_ref[...], 1.0)
    h = jnp.dot(aggf * inv, wl_ref[...], preferred_element_type=jnp.float32)
    h = h + b_ref[...] + jnp.dot(xf, wr_ref[...],
                                 preferred_element_type=jnp.float32)
    h = jnp.maximum(h, 0.0)
    if paired_out:
        out_ref[0] = h[:, :H]
        out_ref[1] = h[:, H:]
    else:
        out_ref[...] = h


def _make_dense(paired_out: bool):
    grid = (N_PAD // BN,)
    pair_spec = pl.BlockSpec((2, BN, H), lambda i: (0, i, 0))
    in_specs = [
        pair_spec,
        pl.BlockSpec((BN, 1), lambda i: (i, 0)),
        pair_spec,
        pl.BlockSpec((D, D), lambda i: (0, 0)),
        pl.BlockSpec((D, D), lambda i: (0, 0)),
        pl.BlockSpec((1, D), lambda i: (0, 0)),
    ]
    if paired_out:
        out_shape = jax.ShapeDtypeStruct((2, N_PAD, H), jnp.float32)
        out_specs = pair_spec
    else:
        out_shape = jax.ShapeDtypeStruct((N_PAD, D), jnp.float32)
        out_specs = pl.BlockSpec((BN, D), lambda i: (i, 0))
    return pl.pallas_call(
        functools.partial(_dense_body, paired_out),
        grid=grid, in_specs=in_specs, out_specs=out_specs,
        out_shape=out_shape)


_make_sc_aggregate = functools.lru_cache(None)(_make_sc_aggregate)
_make_dense = functools.lru_cache(None)(_make_dense)

CHUNK1, GROUPS1 = 64, 8
CHUNK2, GROUPS2 = 80, 4


def kernel(x, edge_index, W1_l, b1, W1_r, W2_l, b2, W2_r):
    src = edge_index[0].astype(jnp.int32)
    dst = edge_index[1].astype(jnp.int32)
    src_p = jnp.concatenate([src, jnp.zeros((E_PAD - E,), jnp.int32)])
    dst_p = jnp.concatenate([dst, jnp.full((E_PAD - E,), N, jnp.int32)])
    src2 = jnp.stack([src_p, src_p + N_PAD])

    def idx4(a, chunk, groups, lead):
        return a.reshape(lead + (16, groups, E_PER_TILE // (groups * chunk),
                                 chunk))

    xt = x.reshape(N, 2, H).transpose(1, 0, 2)
    xt = jnp.pad(xt, ((0, 0), (0, N_PAD - N), (0, 0)))
    x2n = xt.reshape(2 * N_PAD, H)

    agg1, deg, _ = _make_sc_aggregate(True, CHUNK1, GROUPS1)(
        x2n, idx4(src2, CHUNK1, GROUPS1, (2,)), idx4(dst_p, CHUNK1, GROUPS1,
                                                     ()))
    deg_col = deg.reshape(N_PAD, 1)
    h2n = _make_dense(True)(agg1, deg_col, x2n.reshape(2, N_PAD, H),
                            W1_l.T, W1_r.T, b1.reshape(1, D))
    agg2, = _make_sc_aggregate(False, CHUNK2, GROUPS2)(
        h2n.reshape(2 * N_PAD, H), idx4(src2, CHUNK2, GROUPS2, (2,)),
        idx4(dst_p, CHUNK2, GROUPS2, ()))
    out = _make_dense(False)(agg2, deg_col, h2n, W2_l.T, W2_r.T,
                             b2.reshape(1, D))
    return out[:N]


# speedup vs baseline: 1.0102x; 1.0102x over previous
import functools

import jax
import jax.numpy as jnp
from jax import lax
from jax.experimental import pallas as pl
from jax.experimental.pallas import tpu as pltpu
from jax.experimental.pallas import tpu_sc as plsc

N = 10000
E = 160000
D = 256
H = 128
N_PAD = 10240
RPT = N_PAD // 16
GROUPS = 4
E_PER_TILE = 10240
E_PAD = E_PER_TILE * 16
BN = 512
BNR = 400


def _make_sc_aggregate(compute_deg: bool, chunk: int):
    gchunks = E_PER_TILE // (GROUPS * chunk)
    mesh = plsc.VectorSubcoreMesh(core_axis_name="c", subcore_axis_name="s",
                                  num_cores=2, num_subcores=16)
    out_type = [jax.ShapeDtypeStruct((2, N_PAD, H), jnp.float32)]
    scratch = [
        pltpu.VMEM((gchunks, chunk), jnp.int32),
        pltpu.VMEM((gchunks, chunk), jnp.int32),
        pltpu.VMEM((chunk, H), jnp.float32),
        pltpu.VMEM((chunk, H), jnp.float32),
        pltpu.VMEM_SHARED((N_PAD, H), jnp.float32),
        pltpu.SemaphoreType.DMA,
        pltpu.SemaphoreType.DMA,
    ]
    if compute_deg:
        out_type.append(jax.ShapeDtypeStruct((N_PAD,), jnp.float32))
        scratch += [
            pltpu.VMEM((N_PAD,), jnp.float32),
            pltpu.VMEM_SHARED((16, N_PAD), jnp.float32),
        ]

    kchunk = chunk // 16

    def body(x2n, src_idx, dst_idx, *rest):
        if compute_deg:
            (agg_out, deg_out, src_v, dst_v, rows_a, rows_b, agg_sh,
             sem_a, sem_b, hist_v, stage_sh) = rest
        else:
            (agg_out, src_v, dst_v, rows_a, rows_b, agg_sh,
             sem_a, sem_b) = rest

        c = lax.axis_index("c")
        s = lax.axis_index("s")
        base = s * RPT
        zeros16 = jnp.zeros((16,), jnp.float32)
        for r in range(16):
            for k in range(H // 16):
                rows_a[r, pl.ds(k * 16, 16)] = zeros16

        if compute_deg:
            def zero_hist(g, carry):
                hist_v[pl.ds(g * 16, 16)] = zeros16
                return carry
            lax.fori_loop(0, N_PAD // 16, zero_hist, 0)

        def zero_agg(i, carry):
            pltpu.sync_copy(rows_a.at[pl.ds(0, 16)],
                            agg_sh.at[pl.ds(base + i * 16, 16)])
            return carry
        lax.fori_loop(0, RPT // 16, zero_agg, 0)

        plsc.subcore_barrier()

        def hist_chunk(j):
            if not compute_deg:
                return

            @pl.when(c == 0)
            def _():
                iota16 = lax.iota(jnp.int32, 16)
                one16 = jnp.ones((16,), jnp.float32)
                zero16 = jnp.zeros((16,), jnp.float32)
                for k in range(kchunk):
                    dvec = dst_v[j, pl.ds(k * 16, 16)]
                    for l in range(16):
                        idx = dvec[l]
                        wbase = lax.bitwise_and(idx, ~15)
                        lane = lax.bitwise_and(idx, 15)
                        oh = jnp.where(iota16 == lane, one16, zero16)
                        w = hist_v[pl.ds(wbase, 16)]
                        hist_v[pl.ds(wbase, 16)] = w + oh

        def wait(rows, sem):
            pltpu.make_async_copy(x2n.at[pl.ds(0, chunk)], rows, sem).wait()

        def group(g, carry):
            pltpu.sync_copy(src_idx.at[s, g], src_v)
            pltpu.sync_copy(dst_idx.at[s, g], dst_v)

            @pl.when(c == 1)
            def _():
                npad16 = jnp.full((16,), N_PAD, jnp.int32)

                def off(i, carry2):
                    r = lax.shift_right_logical(i, kchunk.bit_length() - 1)
                    k = lax.bitwise_and(i, kchunk - 1)
                    src_v[r, pl.ds(k * 16, 16)] = (
                        src_v[r, pl.ds(k * 16, 16)] + npad16)
                    return carry2
                lax.fori_loop(0, gchunks * kchunk, off, 0)

            pltpu.async_copy(x2n.at[src_v.at[0]], rows_a, sem_a)

            def pair(t, carry2):
                j0 = t * 2
                pltpu.async_copy(x2n.at[src_v.at[j0 + 1]], rows_b, sem_b)
                hist_chunk(j0)
                wait(rows_a, sem_a)
                pltpu.sync_copy(rows_a, agg_sh.at[dst_v.at[j0]], add=True)

                @pl.when(j0 + 2 < gchunks)
                def _():
                    pltpu.async_copy(x2n.at[src_v.at[j0 + 2]], rows_a, sem_a)
                hist_chunk(j0 + 1)
                wait(rows_b, sem_b)
                pltpu.sync_copy(rows_b, agg_sh.at[dst_v.at[j0 + 1]], add=True)
                return carry2
            lax.fori_loop(0, gchunks // 2, pair, 0)
            return carry
        lax.fori_loop(0, GROUPS, group, 0)

        if compute_deg:
            @pl.when(c == 0)
            def _():
                pltpu.sync_copy(hist_v, stage_sh.at[s])
        plsc.subcore_barrier()

        if compute_deg:
            @pl.when(c == 0)
            def _():
                def zero_dsum(g, carry):
                    hist_v[pl.ds(g * 16, 16)] = zeros16
                    return carry
                lax.fori_loop(0, RPT // 16, zero_dsum, 0)
                for t in range(16):
                    pltpu.sync_copy(stage_sh.at[t, pl.ds(base, RPT)],
                                    hist_v.at[pl.ds(RPT, RPT)])

                    def acc(g, carry):
                        hist_v[pl.ds(g * 16, 16)] = (
                            hist_v[pl.ds(g * 16, 16)]
                            + hist_v[pl.ds(RPT + g * 16, 16)])
                        return carry
                    lax.fori_loop(0, RPT // 16, acc, 0)
                pltpu.sync_copy(hist_v.at[pl.ds(0, RPT)],
                                deg_out.at[pl.ds(base, RPT)])

        pltpu.sync_copy(agg_sh.at[pl.ds(base, RPT)],
                        agg_out.at[c, pl.ds(base, RPT)])

    return pl.kernel(body, out_type=out_type, mesh=mesh,
                     scratch_types=scratch)


def _repack_body(x_ref, out_ref):
    out_ref[0] = x_ref[:, :H]
    out_ref[1] = x_ref[:, H:]


def _make_repack():
    return pl.pallas_call(
        _repack_body,
        grid=(N // BNR,),
        in_specs=[pl.BlockSpec((BNR, D), lambda i: (i, 0))],
        out_specs=pl.BlockSpec((2, BNR, H), lambda i: (0, i, 0)),
        out_shape=jax.ShapeDtypeStruct((2, N_PAD, H), jnp.float32))


def _xterm_body(x_ref, wr_ref, b_ref, out_ref):
    out_ref[...] = b_ref[...] + jnp.dot(x_ref[...], wr_ref[...],
                                        preferred_element_type=jnp.float32)


def _make_xterm():
    return pl.pallas_call(
        _xterm_body,
        grid=(N // BNR,),
        in_specs=[pl.BlockSpec((BNR, D), lambda i: (i, 0)),
                  pl.BlockSpec((D, D), lambda i: (0, 0)),
                  pl.BlockSpec((1, D), lambda i: (0, 0))],
        out_specs=pl.BlockSpec((BNR, D), lambda i: (i, 0)),
        out_shape=jax.ShapeDtypeStruct((N_PAD, D), jnp.float32))


def _final1_body(agg_ref, deg_ref, xt_ref, wl_ref, out_ref):
    aggf = jnp.concatenate([agg_ref[0], agg_ref[1]], axis=1)
    inv = 1.0 / jnp.maximum(deg_ref[...], 1.0)
    h = jnp.dot(aggf * inv, wl_ref[...], preferred_element_type=jnp.float32)
    h = jnp.maximum(h + xt_ref[...], 0.0)
    out_ref[0] = h[:, :H]
    out_ref[1] = h[:, H:]


def _make_final1():
    pair_spec = pl.BlockSpec((2, BN, H), lambda i: (0, i, 0))
    return pl.pallas_call(
        _final1_body,
        grid=(N_PAD // BN,),
        in_specs=[pair_spec,
                  pl.BlockSpec((BN, 1), lambda i: (i, 0)),
                  pl.BlockSpec((BN, D), lambda i: (i, 0)),
                  pl.BlockSpec((D, D), lambda i: (0, 0))],
        out_specs=pair_spec,
        out_shape=jax.ShapeDtypeStruct((2, N_PAD, H), jnp.float32))


def _final2_body(agg_ref, deg_ref, xr_ref, wl_ref, wr_ref, b_ref, out_ref):
    aggf = jnp.concatenate([agg_ref[0], agg_ref[1]], axis=1)
    xf = jnp.concatenate([xr_ref[0], xr_ref[1]], axis=1)
    inv = 1.0 / jnp.maximum(deg_ref[...], 1.0)
    h = jnp.dot(aggf * inv, wl_ref[...], preferred_element_type=jnp.float32)
    h = h + b_ref[...] + jnp.dot(xf, wr_ref[...],
                                 preferred_element_type=jnp.float32)
    out_ref[...] = jnp.maximum(h, 0.0)


def _make_final2():
    pair_spec = pl.BlockSpec((2, BN, H), lambda i: (0, i, 0))
    return pl.pallas_call(
        _final2_body,
        grid=(N_PAD // BN,),
        in_specs=[pair_spec,
                  pl.BlockSpec((BN, 1), lambda i: (i, 0)),
                  pair_spec,
                  pl.BlockSpec((D, D), lambda i: (0, 0)),
                  pl.BlockSpec((D, D), lambda i: (0, 0)),
                  pl.BlockSpec((1, D), lambda i: (0, 0))],
        out_specs=pl.BlockSpec((BN, D), lambda i: (i, 0)),
        out_shape=jax.ShapeDtypeStruct((N_PAD, D), jnp.float32))


_make_sc_aggregate = functools.lru_cache(None)(_make_sc_aggregate)
_make_repack = functools.lru_cache(None)(_make_repack)
_make_xterm = functools.lru_cache(None)(_make_xterm)
_make_final1 = functools.lru_cache(None)(_make_final1)
_make_final2 = functools.lru_cache(None)(_make_final2)

CHUNK1 = 64
CHUNK2 = 128


def kernel(x, edge_index, W1_l, b1, W1_r, W2_l, b2, W2_r):
    src = edge_index[0].astype(jnp.int32)
    dst = edge_index[1].astype(jnp.int32)
    src_p = jnp.concatenate([src, jnp.zeros((E_PAD - E,), jnp.int32)])
    dst_p = jnp.concatenate([dst, jnp.full((E_PAD - E,), N, jnp.int32)])

    def idx4(a, chunk):
        return a.reshape(16, GROUPS, E_PER_TILE // (GROUPS * chunk), chunk)

    x2n = _make_repack()(x)
    xterm = _make_xterm()(x, W1_r.T, b1.reshape(1, D))

    agg1, deg = _make_sc_aggregate(True, CHUNK1)(
        x2n.reshape(2 * N_PAD, H), idx4(src_p, CHUNK1), idx4(dst_p, CHUNK1))
    deg_col = deg.reshape(N_PAD, 1)
    h2n = _make_final1()(agg1, deg_col, xterm, W1_l.T)
    agg2, = _make_sc_aggregate(False, CHUNK2)(
        h2n.reshape(2 * N_PAD, H), idx4(src_p, CHUNK2), idx4(dst_p, CHUNK2))
    out = _make_final2()(agg2, deg_col, h2n, W2_l.T, W2_r.T,
                         b2.reshape(1, D))
    return out[:N]


# speedup vs baseline: 1.0159x; 1.0056x over previous
import functools

import jax
import jax.numpy as jnp
from jax import lax
from jax.experimental import pallas as pl
from jax.experimental.pallas import tpu as pltpu
from jax.experimental.pallas import tpu_sc as plsc

N = 10000
E = 160000
D = 256
H = 128
N_PAD = 10240
RPT = N_PAD // 16
GROUPS = 4
E_PER_TILE = 10240
E_PAD = E_PER_TILE * 16
BN = 512
BNR = 400


def _make_sc_aggregate(compute_deg: bool, chunk: int):
    gchunks = E_PER_TILE // (GROUPS * chunk)
    mesh = plsc.VectorSubcoreMesh(core_axis_name="c", subcore_axis_name="s",
                                  num_cores=2, num_subcores=16)
    out_type = [jax.ShapeDtypeStruct((2, N_PAD, H), jnp.float32)]
    scratch = [
        pltpu.VMEM((gchunks, chunk), jnp.int32),
        pltpu.VMEM((gchunks, chunk), jnp.int32),
        pltpu.VMEM((chunk, H), jnp.float32),
        pltpu.VMEM((chunk, H), jnp.float32),
        pltpu.VMEM_SHARED((N_PAD, H), jnp.float32),
        pltpu.SemaphoreType.DMA,
        pltpu.SemaphoreType.DMA,
    ]
    if compute_deg:
        out_type.append(jax.ShapeDtypeStruct((N_PAD,), jnp.float32))
        scratch += [
            pltpu.VMEM((N_PAD,), jnp.float32),
            pltpu.VMEM_SHARED((16, N_PAD), jnp.float32),
        ]

    kchunk = chunk // 16

    def body(x2n, src_idx0, src_idx1, dst_idx, *rest):
        if compute_deg:
            (agg_out, deg_out, src_v, dst_v, rows_a, rows_b, agg_sh,
             sem_a, sem_b, hist_v, stage_sh) = rest
        else:
            (agg_out, src_v, dst_v, rows_a, rows_b, agg_sh,
             sem_a, sem_b) = rest

        c = lax.axis_index("c")
        s = lax.axis_index("s")
        base = s * RPT
        zeros16 = jnp.zeros((16,), jnp.float32)
        for r in range(16):
            for k in range(H // 16):
                rows_a[r, pl.ds(k * 16, 16)] = zeros16

        if compute_deg:
            def zero_hist(g, carry):
                hist_v[pl.ds(g * 16, 16)] = zeros16
                return carry
            lax.fori_loop(0, N_PAD // 16, zero_hist, 0)

        def zero_agg(i, carry):
            pltpu.sync_copy(rows_a.at[pl.ds(0, 16)],
                            agg_sh.at[pl.ds(base + i * 16, 16)])
            return carry
        lax.fori_loop(0, RPT // 16, zero_agg, 0)

        plsc.subcore_barrier()

        def hist_chunk(j):
            if not compute_deg:
                return

            @pl.when(c == 0)
            def _():
                iota16 = lax.iota(jnp.int32, 16)
                one16 = jnp.ones((16,), jnp.float32)
                zero16 = jnp.zeros((16,), jnp.float32)
                for k in range(kchunk):
                    dvec = dst_v[j, pl.ds(k * 16, 16)]
                    for l in range(16):
                        idx = dvec[l]
                        wbase = lax.bitwise_and(idx, ~15)
                        lane = lax.bitwise_and(idx, 15)
                        oh = jnp.where(iota16 == lane, one16, zero16)
                        w = hist_v[pl.ds(wbase, 16)]
                        hist_v[pl.ds(wbase, 16)] = w + oh

        def wait(rows, sem):
            pltpu.make_async_copy(x2n.at[pl.ds(0, chunk)], rows, sem).wait()

        def group(g, carry):
            @pl.when(c == 0)
            def _():
                pltpu.sync_copy(src_idx0.at[s, g], src_v)

            @pl.when(c == 1)
            def _():
                pltpu.sync_copy(src_idx1.at[s, g], src_v)
            pltpu.sync_copy(dst_idx.at[s, g], dst_v)

            pltpu.async_copy(x2n.at[src_v.at[0]], rows_a, sem_a)

            def pair(t, carry2):
                j0 = t * 2
                pltpu.async_copy(x2n.at[src_v.at[j0 + 1]], rows_b, sem_b)
                hist_chunk(j0)
                wait(rows_a, sem_a)
                pltpu.sync_copy(rows_a, agg_sh.at[dst_v.at[j0]], add=True)

                @pl.when(j0 + 2 < gchunks)
                def _():
                    pltpu.async_copy(x2n.at[src_v.at[j0 + 2]], rows_a, sem_a)
                hist_chunk(j0 + 1)
                wait(rows_b, sem_b)
                pltpu.sync_copy(rows_b, agg_sh.at[dst_v.at[j0 + 1]], add=True)
                return carry2
            lax.fori_loop(0, gchunks // 2, pair, 0)
            return carry
        lax.fori_loop(0, GROUPS, group, 0)

        if compute_deg:
            @pl.when(c == 0)
            def _():
                pltpu.sync_copy(hist_v, stage_sh.at[s])
        plsc.subcore_barrier()

        if compute_deg:
            @pl.when(c == 0)
            def _():
                def zero_dsum(g, carry):
                    hist_v[pl.ds(g * 16, 16)] = zeros16
                    return carry
                lax.fori_loop(0, RPT // 16, zero_dsum, 0)
                for t in range(16):
                    pltpu.sync_copy(stage_sh.at[t, pl.ds(base, RPT)],
                                    hist_v.at[pl.ds(RPT, RPT)])

                    def acc(g, carry):
                        hist_v[pl.ds(g * 16, 16)] = (
                            hist_v[pl.ds(g * 16, 16)]
                            + hist_v[pl.ds(RPT + g * 16, 16)])
                        return carry
                    lax.fori_loop(0, RPT // 16, acc, 0)
                pltpu.sync_copy(hist_v.at[pl.ds(0, RPT)],
                                deg_out.at[pl.ds(base, RPT)])

        pltpu.sync_copy(agg_sh.at[pl.ds(base, RPT)],
                        agg_out.at[c, pl.ds(base, RPT)])

    return pl.kernel(body, out_type=out_type, mesh=mesh,
                     scratch_types=scratch)


def _repack_body(x_ref, out_ref):
    out_ref[0] = x_ref[:, :H]
    out_ref[1] = x_ref[:, H:]


def _make_repack():
    return pl.pallas_call(
        _repack_body,
        grid=(N // BNR,),
        in_specs=[pl.BlockSpec((BNR, D), lambda i: (i, 0))],
        out_specs=pl.BlockSpec((2, BNR, H), lambda i: (0, i, 0)),
        out_shape=jax.ShapeDtypeStruct((2, N_PAD, H), jnp.float32))


def _xterm_body(x_ref, wr_ref, b_ref, out_ref):
    out_ref[...] = b_ref[...] + jnp.dot(x_ref[...], wr_ref[...],
                                        preferred_element_type=jnp.float32)


def _make_xterm():
    return pl.pallas_call(
        _xterm_body,
        grid=(N // BNR,),
        in_specs=[pl.BlockSpec((BNR, D), lambda i: (i, 0)),
                  pl.BlockSpec((D, D), lambda i: (0, 0)),
                  pl.BlockSpec((1, D), lambda i: (0, 0))],
        out_specs=pl.BlockSpec((BNR, D), lambda i: (i, 0)),
        out_shape=jax.ShapeDtypeStruct((N_PAD, D), jnp.float32))


def _final1_body(agg_ref, deg_ref, xt_ref, wl_ref, out_ref):
    aggf = jnp.concatenate([agg_ref[0], agg_ref[1]], axis=1)
    inv = 1.0 / jnp.maximum(deg_ref[...], 1.0)
    h = jnp.dot(aggf * inv, wl_ref[...], preferred_element_type=jnp.float32)
    h = jnp.maximum(h + xt_ref[...], 0.0)
    out_ref[0] = h[:, :H]
    out_ref[1] = h[:, H:]


def _make_final1():
    pair_spec = pl.BlockSpec((2, BN, H), lambda i: (0, i, 0))
    return pl.pallas_call(
        _final1_body,
        grid=(N_PAD // BN,),
        in_specs=[pair_spec,
                  pl.BlockSpec((BN, 1), lambda i: (i, 0)),
                  pl.BlockSpec((BN, D), lambda i: (i, 0)),
                  pl.BlockSpec((D, D), lambda i: (0, 0))],
        out_specs=pair_spec,
        out_shape=jax.ShapeDtypeStruct((2, N_PAD, H), jnp.float32))


def _final2_body(agg_ref, deg_ref, xr_ref, wl_ref, wr_ref, b_ref, out_ref):
    aggf = jnp.concatenate([agg_ref[0], agg_ref[1]], axis=1)
    xf = jnp.concatenate([xr_ref[0], xr_ref[1]], axis=1)
    inv = 1.0 / jnp.maximum(deg_ref[...], 1.0)
    h = jnp.dot(aggf * inv, wl_ref[...], preferred_element_type=jnp.float32)
    h = h + b_ref[...] + jnp.dot(xf, wr_ref[...],
                                 preferred_element_type=jnp.float32)
    out_ref[...] = jnp.maximum(h, 0.0)


def _make_final2():
    pair_spec = pl.BlockSpec((2, BN, H), lambda i: (0, i, 0))
    return pl.pallas_call(
        _final2_body,
        grid=(N_PAD // BN,),
        in_specs=[pair_spec,
                  pl.BlockSpec((BN, 1), lambda i: (i, 0)),
                  pair_spec,
                  pl.BlockSpec((D, D), lambda i: (0, 0)),
                  pl.BlockSpec((D, D), lambda i: (0, 0)),
                  pl.BlockSpec((1, D), lambda i: (0, 0))],
        out_specs=pl.BlockSpec((BN, D), lambda i: (i, 0)),
        out_shape=jax.ShapeDtypeStruct((N_PAD, D), jnp.float32))


_make_sc_aggregate = functools.lru_cache(None)(_make_sc_aggregate)
_make_repack = functools.lru_cache(None)(_make_repack)
_make_xterm = functools.lru_cache(None)(_make_xterm)
_make_final1 = functools.lru_cache(None)(_make_final1)
_make_final2 = functools.lru_cache(None)(_make_final2)

CHUNK1 = 64
CHUNK2 = 128


def kernel(x, edge_index, W1_l, b1, W1_r, W2_l, b2, W2_r):
    src = edge_index[0].astype(jnp.int32)
    dst = edge_index[1].astype(jnp.int32)
    src_p = jnp.concatenate([src, jnp.zeros((E_PAD - E,), jnp.int32)])
    dst_p = jnp.concatenate([dst, jnp.full((E_PAD - E,), N, jnp.int32)])

    def idx4(a, chunk):
        return a.reshape(16, GROUPS, E_PER_TILE // (GROUPS * chunk), chunk)

    x2n = _make_repack()(x)
    xterm = _make_xterm()(x, W1_r.T, b1.reshape(1, D))

    src_q = src_p + N_PAD
    agg1, deg = _make_sc_aggregate(True, CHUNK1)(
        x2n.reshape(2 * N_PAD, H), idx4(src_p, CHUNK1), idx4(src_q, CHUNK1),
        idx4(dst_p, CHUNK1))
    deg_col = deg.reshape(N_PAD, 1)
    h2n = _make_final1()(agg1, deg_col, xterm, W1_l.T)
    agg2, = _make_sc_aggregate(False, CHUNK2)(
        h2n.reshape(2 * N_PAD, H), idx4(src_p, CHUNK2), idx4(src_q, CHUNK2),
        idx4(dst_p, CHUNK2))
    out = _make_final2()(agg2, deg_col, h2n, W2_l.T, W2_r.T,
                         b2.reshape(1, D))
    return out[:N]


# speedup vs baseline: 1.0566x; 1.0401x over previous
import functools

import jax
import jax.numpy as jnp
from jax import lax
from jax.experimental import pallas as pl
from jax.experimental.pallas import tpu as pltpu
from jax.experimental.pallas import tpu_sc as plsc

N = 10000
E = 160000
D = 256
H = 128
N_PAD = 10240
RPT = N_PAD // 16
GROUPS = 4
E_PER_TILE = 10240
E_PAD = E_PER_TILE * 16
BN = 512
BNR = 400


def _make_sc_aggregate(compute_deg: bool, chunk: int):
    gchunks = E_PER_TILE // (GROUPS * chunk)
    mesh = plsc.VectorSubcoreMesh(core_axis_name="c", subcore_axis_name="s",
                                  num_cores=2, num_subcores=16)
    out_type = [jax.ShapeDtypeStruct((2, N_PAD, H), jnp.float32)]
    scratch = [
        pltpu.VMEM((gchunks, chunk), jnp.int32),
        pltpu.VMEM((gchunks, chunk), jnp.int32),
        pltpu.VMEM((chunk, H), jnp.float32),
        pltpu.VMEM((chunk, H), jnp.float32),
        pltpu.VMEM_SHARED((N_PAD, H), jnp.float32),
        pltpu.SemaphoreType.DMA,
        pltpu.SemaphoreType.DMA,
    ]
    if compute_deg:
        out_type.append(jax.ShapeDtypeStruct((N_PAD,), jnp.float32))
        scratch += [
            pltpu.VMEM((N_PAD,), jnp.float32),
            pltpu.VMEM_SHARED((16, N_PAD), jnp.float32),
        ]

    kchunk = chunk // 16

    def body(x2n, src_idx0, src_idx1, dst_idx, *rest):
        if compute_deg:
            (agg_out, deg_out, src_v, dst_v, rows_a, rows_b, agg_sh,
             sem_a, sem_b, hist_v, stage_sh) = rest
        else:
            (agg_out, src_v, dst_v, rows_a, rows_b, agg_sh,
             sem_a, sem_b) = rest

        c = lax.axis_index("c")
        s = lax.axis_index("s")
        base = s * RPT
        zeros16 = jnp.zeros((16,), jnp.float32)
        for r in range(16):
            for k in range(H // 16):
                rows_a[r, pl.ds(k * 16, 16)] = zeros16

        if compute_deg:
            def zero_hist(g, carry):
                hist_v[pl.ds(g * 16, 16)] = zeros16
                return carry
            lax.fori_loop(0, N_PAD // 16, zero_hist, 0)

        def zero_agg(i, carry):
            pltpu.sync_copy(rows_a.at[pl.ds(0, 16)],
                            agg_sh.at[pl.ds(base + i * 16, 16)])
            return carry
        lax.fori_loop(0, RPT // 16, zero_agg, 0)

        plsc.subcore_barrier()

        def hist_chunk(j):
            if not compute_deg:
                return

            @pl.when(c == 0)
            def _():
                iota16 = lax.iota(jnp.int32, 16)
                one16 = jnp.ones((16,), jnp.float32)
                zero16 = jnp.zeros((16,), jnp.float32)
                for k in range(kchunk):
                    dvec = dst_v[j, pl.ds(k * 16, 16)]
                    for l in range(16):
                        idx = dvec[l]
                        wbase = lax.bitwise_and(idx, ~15)
                        lane = lax.bitwise_and(idx, 15)
                        oh = jnp.where(iota16 == lane, one16, zero16)
                        w = hist_v[pl.ds(wbase, 16)]
                        hist_v[pl.ds(wbase, 16)] = w + oh

        def wait(rows, sem):
            pltpu.make_async_copy(x2n.at[pl.ds(0, chunk)], rows, sem).wait()

        def group(g, carry):
            @pl.when(c == 0)
            def _():
                pltpu.sync_copy(src_idx0.at[s, g], src_v)

            @pl.when(c == 1)
            def _():
                pltpu.sync_copy(src_idx1.at[s, g], src_v)
            pltpu.sync_copy(dst_idx.at[s, g], dst_v)

            pltpu.async_copy(x2n.at[src_v.at[0]], rows_a, sem_a)

            def pair(t, carry2):
                j0 = t * 2
                pltpu.async_copy(x2n.at[src_v.at[j0 + 1]], rows_b, sem_b)
                hist_chunk(j0)
                wait(rows_a, sem_a)
                pltpu.sync_copy(rows_a, agg_sh.at[dst_v.at[j0]], add=True)

                @pl.when(j0 + 2 < gchunks)
                def _():
                    pltpu.async_copy(x2n.at[src_v.at[j0 + 2]], rows_a, sem_a)
                hist_chunk(j0 + 1)
                wait(rows_b, sem_b)
                pltpu.sync_copy(rows_b, agg_sh.at[dst_v.at[j0 + 1]], add=True)
                return carry2
            lax.fori_loop(0, gchunks // 2, pair, 0)
            return carry
        lax.fori_loop(0, GROUPS, group, 0)

        if compute_deg:
            @pl.when(c == 0)
            def _():
                pltpu.sync_copy(hist_v, stage_sh.at[s])
        plsc.subcore_barrier()

        if compute_deg:
            @pl.when(c == 0)
            def _():
                def zero_dsum(g, carry):
                    hist_v[pl.ds(g * 16, 16)] = zeros16
                    return carry
                lax.fori_loop(0, RPT // 16, zero_dsum, 0)
                for t in range(16):
                    pltpu.sync_copy(stage_sh.at[t, pl.ds(base, RPT)],
                                    hist_v.at[pl.ds(RPT, RPT)])

                    def acc(g, carry):
                        hist_v[pl.ds(g * 16, 16)] = (
                            hist_v[pl.ds(g * 16, 16)]
                            + hist_v[pl.ds(RPT + g * 16, 16)])
                        return carry
                    lax.fori_loop(0, RPT // 16, acc, 0)
                pltpu.sync_copy(hist_v.at[pl.ds(0, RPT)],
                                deg_out.at[pl.ds(base, RPT)])

        pltpu.sync_copy(agg_sh.at[pl.ds(base, RPT)],
                        agg_out.at[c, pl.ds(base, RPT)])

    return pl.kernel(body, out_type=out_type, mesh=mesh,
                     scratch_types=scratch)


def _repack_body(x_ref, out_ref):
    out_ref[0] = x_ref[:, :H]
    out_ref[1] = x_ref[:, H:]


def _make_repack():
    return pl.pallas_call(
        _repack_body,
        grid=(N // BNR,),
        in_specs=[pl.BlockSpec((BNR, D), lambda i: (i, 0))],
        out_specs=pl.BlockSpec((2, BNR, H), lambda i: (0, i, 0)),
        out_shape=jax.ShapeDtypeStruct((2, N_PAD, H), jnp.float32))


def _dense1_body(agg_ref, deg_ref, x_ref, wl_ref, wr_ref, b_ref, out_ref):
    aggf = jnp.concatenate([agg_ref[0], agg_ref[1]], axis=1)
    inv = 1.0 / jnp.maximum(deg_ref[...], 1.0)
    h = jnp.dot(aggf * inv, wl_ref[...], preferred_element_type=jnp.float32)
    h = h + b_ref[...] + jnp.dot(x_ref[...], wr_ref[...],
                                 preferred_element_type=jnp.float32)
    h = jnp.maximum(h, 0.0)
    out_ref[0] = h[:, :H]
    out_ref[1] = h[:, H:]


def _make_dense1():
    pair_spec = pl.BlockSpec((2, BNR, H), lambda i: (0, i, 0))
    return pl.pallas_call(
        _dense1_body,
        grid=(N // BNR,),
        in_specs=[pair_spec,
                  pl.BlockSpec((BNR, 1), lambda i: (i, 0)),
                  pl.BlockSpec((BNR, D), lambda i: (i, 0)),
                  pl.BlockSpec((D, D), lambda i: (0, 0)),
                  pl.BlockSpec((D, D), lambda i: (0, 0)),
                  pl.BlockSpec((1, D), lambda i: (0, 0))],
        out_specs=pair_spec,
        out_shape=jax.ShapeDtypeStruct((2, N_PAD, H), jnp.float32))


def _dense2_body(agg_ref, deg_ref, xr_ref, wl_ref, wr_ref, b_ref, out_ref):
    aggf = jnp.concatenate([agg_ref[0], agg_ref[1]], axis=1)
    xf = jnp.concatenate([xr_ref[0], xr_ref[1]], axis=1)
    inv = 1.0 / jnp.maximum(deg_ref[...], 1.0)
    h = jnp.dot(aggf * inv, wl_ref[...], preferred_element_type=jnp.float32)
    h = h + b_ref[...] + jnp.dot(xf, wr_ref[...],
                                 preferred_element_type=jnp.float32)
    out_ref[...] = jnp.maximum(h, 0.0)


def _make_dense2():
    pair_spec = pl.BlockSpec((2, BNR, H), lambda i: (0, i, 0))
    return pl.pallas_call(
        _dense2_body,
        grid=(N // BNR,),
        in_specs=[pair_spec,
                  pl.BlockSpec((BNR, 1), lambda i: (i, 0)),
                  pair_spec,
                  pl.BlockSpec((D, D), lambda i: (0, 0)),
                  pl.BlockSpec((D, D), lambda i: (0, 0)),
                  pl.BlockSpec((1, D), lambda i: (0, 0))],
        out_specs=pl.BlockSpec((BNR, D), lambda i: (i, 0)),
        out_shape=jax.ShapeDtypeStruct((N, D), jnp.float32))


_make_sc_aggregate = functools.lru_cache(None)(_make_sc_aggregate)
_make_repack = functools.lru_cache(None)(_make_repack)
_make_dense1 = functools.lru_cache(None)(_make_dense1)
_make_dense2 = functools.lru_cache(None)(_make_dense2)

CHUNK1 = 64
CHUNK2 = 128


def kernel(x, edge_index, W1_l, b1, W1_r, W2_l, b2, W2_r):
    src = edge_index[0].astype(jnp.int32)
    dst = edge_index[1].astype(jnp.int32)
    src_p = jnp.concatenate([src, jnp.zeros((E_PAD - E,), jnp.int32)])
    dst_p = jnp.concatenate([dst, jnp.full((E_PAD - E,), N, jnp.int32)])

    def idx4(a, chunk):
        return a.reshape(16, GROUPS, E_PER_TILE // (GROUPS * chunk), chunk)

    x2n = _make_repack()(x)

    src_q = src_p + N_PAD
    agg1, deg = _make_sc_aggregate(True, CHUNK1)(
        x2n.reshape(2 * N_PAD, H), idx4(src_p, CHUNK1), idx4(src_q, CHUNK1),
        idx4(dst_p, CHUNK1))
    deg_col = deg.reshape(N_PAD, 1)
    h2n = _make_dense1()(agg1, deg_col, x, W1_l.T, W1_r.T, b1.reshape(1, D))
    agg2, = _make_sc_aggregate(False, CHUNK2)(
        h2n.reshape(2 * N_PAD, H), idx4(src_p, CHUNK2), idx4(src_q, CHUNK2),
        idx4(dst_p, CHUNK2))
    return _make_dense2()(agg2, deg_col, h2n, W2_l.T, W2_r.T,
                          b2.reshape(1, D))


# speedup vs baseline: 1.0569x; 1.0002x over previous
import functools

import jax
import jax.numpy as jnp
from jax import lax
from jax.experimental import pallas as pl
from jax.experimental.pallas import tpu as pltpu
from jax.experimental.pallas import tpu_sc as plsc

N = 10000
E = 160000
D = 256
H = 128
N_PAD = 10240
RPT = N_PAD // 16
GROUPS = 4
E_PER_TILE = 10240
E_PAD = E_PER_TILE * 16
BNR = 400


def _make_sc_aggregate(compute_deg: bool, chunk: int):
    gchunks = E_PER_TILE // (GROUPS * chunk)
    mesh = plsc.VectorSubcoreMesh(core_axis_name="c", subcore_axis_name="s",
                                  num_cores=2, num_subcores=16)
    out_type = [jax.ShapeDtypeStruct((2, N_PAD, H), jnp.float32)]
    scratch = [
        pltpu.VMEM((gchunks, chunk), jnp.int32),
        pltpu.VMEM((gchunks, chunk), jnp.int32),
        pltpu.VMEM((chunk, H), jnp.float32),
        pltpu.VMEM((chunk, H), jnp.float32),
        pltpu.VMEM_SHARED((N_PAD, H), jnp.float32),
        pltpu.SemaphoreType.DMA,
        pltpu.SemaphoreType.DMA,
    ]
    if compute_deg:
        out_type.append(jax.ShapeDtypeStruct((N_PAD,), jnp.float32))
        scratch += [
            pltpu.VMEM((N_PAD,), jnp.float32),
            pltpu.VMEM_SHARED((16, N_PAD), jnp.float32),
        ]

    kchunk = chunk // 16

    def body(x2n, src_idx0, src_idx1, dst_idx, *rest):
        if compute_deg:
            (agg_out, deg_out, src_v, dst_v, rows_a, rows_b, agg_sh,
             sem_a, sem_b, hist_v, stage_sh) = rest
        else:
            (agg_out, src_v, dst_v, rows_a, rows_b, agg_sh,
             sem_a, sem_b) = rest

        c = lax.axis_index("c")
        s = lax.axis_index("s")
        base = s * RPT
        zeros16 = jnp.zeros((16,), jnp.float32)
        for r in range(16):
            for k in range(H // 16):
                rows_a[r, pl.ds(k * 16, 16)] = zeros16

        if compute_deg:
            def zero_hist(g, carry):
                hist_v[pl.ds(g * 16, 16)] = zeros16
                return carry
            lax.fori_loop(0, N_PAD // 16, zero_hist, 0)

        def zero_agg(i, carry):
            pltpu.sync_copy(rows_a.at[pl.ds(0, 16)],
                            agg_sh.at[pl.ds(base + i * 16, 16)])
            return carry
        lax.fori_loop(0, RPT // 16, zero_agg, 0)

        plsc.subcore_barrier()

        def hist_chunk(j):
            if not compute_deg:
                return

            @pl.when(c == 0)
            def _():
                iota16 = lax.iota(jnp.int32, 16)
                one16 = jnp.ones((16,), jnp.float32)
                zero16 = jnp.zeros((16,), jnp.float32)
                for k in range(kchunk):
                    dvec = dst_v[j, pl.ds(k * 16, 16)]
                    for l in range(16):
                        idx = dvec[l]
                        wbase = lax.bitwise_and(idx, ~15)
                        lane = lax.bitwise_and(idx, 15)
                        oh = jnp.where(iota16 == lane, one16, zero16)
                        w = hist_v[pl.ds(wbase, 16)]
                        hist_v[pl.ds(wbase, 16)] = w + oh

        def wait(rows, sem):
            pltpu.make_async_copy(x2n.at[pl.ds(0, chunk)], rows, sem).wait()

        def group(g, carry):
            @pl.when(c == 0)
            def _():
                pltpu.sync_copy(src_idx0.at[s, g], src_v)

            @pl.when(c == 1)
            def _():
                pltpu.sync_copy(src_idx1.at[s, g], src_v)
            pltpu.sync_copy(dst_idx.at[s, g], dst_v)

            pltpu.async_copy(x2n.at[src_v.at[0]], rows_a, sem_a)

            def pair(t, carry2):
                j0 = t * 2
                pltpu.async_copy(x2n.at[src_v.at[j0 + 1]], rows_b, sem_b)
                hist_chunk(j0)
                wait(rows_a, sem_a)
                pltpu.sync_copy(rows_a, agg_sh.at[dst_v.at[j0]], add=True)

                @pl.when(j0 + 2 < gchunks)
                def _():
                    pltpu.async_copy(x2n.at[src_v.at[j0 + 2]], rows_a, sem_a)
                hist_chunk(j0 + 1)
                wait(rows_b, sem_b)
                pltpu.sync_copy(rows_b, agg_sh.at[dst_v.at[j0 + 1]], add=True)
                return carry2
            lax.fori_loop(0, gchunks // 2, pair, 0)
            return carry
        lax.fori_loop(0, GROUPS, group, 0)

        if compute_deg:
            @pl.when(c == 0)
            def _():
                pltpu.sync_copy(hist_v, stage_sh.at[s])
        plsc.subcore_barrier()

        if compute_deg:
            @pl.when(c == 0)
            def _():
                def zero_dsum(g, carry):
                    hist_v[pl.ds(g * 16, 16)] = zeros16
                    return carry
                lax.fori_loop(0, RPT // 16, zero_dsum, 0)
                for t in range(16):
                    pltpu.sync_copy(stage_sh.at[t, pl.ds(base, RPT)],
                                    hist_v.at[pl.ds(RPT, RPT)])

                    def acc(g, carry):
                        hist_v[pl.ds(g * 16, 16)] = (
                            hist_v[pl.ds(g * 16, 16)]
                            + hist_v[pl.ds(RPT + g * 16, 16)])
                        return carry
                    lax.fori_loop(0, RPT // 16, acc, 0)
                pltpu.sync_copy(hist_v.at[pl.ds(0, RPT)],
                                deg_out.at[pl.ds(base, RPT)])

        pltpu.sync_copy(agg_sh.at[pl.ds(base, RPT)],
                        agg_out.at[c, pl.ds(base, RPT)])

    return pl.kernel(body, out_type=out_type, mesh=mesh,
                     scratch_types=scratch)


def _repack_body(x_ref, out_ref):
    out_ref[0] = x_ref[:, :H]
    out_ref[1] = x_ref[:, H:]


def _make_repack():
    return pl.pallas_call(
        _repack_body,
        grid=(N // BNR,),
        in_specs=[pl.BlockSpec((BNR, D), lambda i: (i, 0))],
        out_specs=pl.BlockSpec((2, BNR, H), lambda i: (0, i, 0)),
        out_shape=jax.ShapeDtypeStruct((2, N_PAD, H), jnp.float32))


def _dense1_body(agg_ref, deg_ref, x_ref, wl_ref, wr_ref, b_ref, out_ref):
    aggf = jnp.concatenate([agg_ref[0], agg_ref[1]], axis=1)
    inv = 1.0 / jnp.maximum(deg_ref[...], 1.0)
    h = jnp.dot(aggf * inv, wl_ref[...], preferred_element_type=jnp.float32)
    h = h + b_ref[...] + jnp.dot(x_ref[...], wr_ref[...],
                                 preferred_element_type=jnp.float32)
    h = jnp.maximum(h, 0.0)
    out_ref[0] = h[:, :H]
    out_ref[1] = h[:, H:]


def _make_dense1():
    pair_spec = pl.BlockSpec((2, BNR, H), lambda i: (0, i, 0))
    return pl.pallas_call(
        _dense1_body,
        grid=(N // BNR,),
        in_specs=[pair_spec,
                  pl.BlockSpec((BNR, 1), lambda i: (i, 0)),
                  pl.BlockSpec((BNR, D), lambda i: (i, 0)),
                  pl.BlockSpec((D, D), lambda i: (0, 0)),
                  pl.BlockSpec((D, D), lambda i: (0, 0)),
                  pl.BlockSpec((1, D), lambda i: (0, 0))],
        out_specs=pair_spec,
        out_shape=jax.ShapeDtypeStruct((2, N_PAD, H), jnp.float32))


def _dense2_body(agg_ref, deg_ref, xr_ref, wl_ref, wr_ref, b_ref, out_ref):
    aggf = jnp.concatenate([agg_ref[0], agg_ref[1]], axis=1)
    xf = jnp.concatenate([xr_ref[0], xr_ref[1]], axis=1)
    inv = 1.0 / jnp.maximum(deg_ref[...], 1.0)
    h = jnp.dot(aggf * inv, wl_ref[...], preferred_element_type=jnp.float32)
    h = h + b_ref[...] + jnp.dot(xf, wr_ref[...],
                                 preferred_element_type=jnp.float32)
    out_ref[...] = jnp.maximum(h, 0.0)


def _make_dense2():
    pair_spec = pl.BlockSpec((2, BNR, H), lambda i: (0, i, 0))
    return pl.pallas_call(
        _dense2_body,
        grid=(N // BNR,),
        in_specs=[pair_spec,
                  pl.BlockSpec((BNR, 1), lambda i: (i, 0)),
                  pair_spec,
                  pl.BlockSpec((D, D), lambda i: (0, 0)),
                  pl.BlockSpec((D, D), lambda i: (0, 0)),
                  pl.BlockSpec((1, D), lambda i: (0, 0))],
        out_specs=pl.BlockSpec((BNR, D), lambda i: (i, 0)),
        out_shape=jax.ShapeDtypeStruct((N, D), jnp.float32))


_make_sc_aggregate = functools.lru_cache(None)(_make_sc_aggregate)
_make_repack = functools.lru_cache(None)(_make_repack)
_make_dense1 = functools.lru_cache(None)(_make_dense1)
_make_dense2 = functools.lru_cache(None)(_make_dense2)

CHUNK1 = 64
CHUNK2 = 128


def kernel(x, edge_index, W1_l, b1, W1_r, W2_l, b2, W2_r):
    src = edge_index[0].astype(jnp.int32)
    dst = edge_index[1].astype(jnp.int32)
    src_p = jnp.concatenate([src, jnp.zeros((E_PAD - E,), jnp.int32)])
    dst_p = jnp.concatenate([dst, jnp.full((E_PAD - E,), N, jnp.int32)])

    def idx4(a, chunk):
        return a.reshape(16, GROUPS, E_PER_TILE // (GROUPS * chunk), chunk)

    x2n = _make_repack()(x)

    src_q = src_p + N_PAD
    agg1, deg = _make_sc_aggregate(True, CHUNK1)(
        x2n.reshape(2 * N_PAD, H), idx4(src_p, CHUNK1), idx4(src_q, CHUNK1),
        idx4(dst_p, CHUNK1))
    deg_col = deg.reshape(N_PAD, 1)
    h2n = _make_dense1()(agg1, deg_col, x, W1_l.T, W1_r.T, b1.reshape(1, D))
    agg2, = _make_sc_aggregate(False, CHUNK2)(
        h2n.reshape(2 * N_PAD, H), idx4(src_p, CHUNK2), idx4(src_q, CHUNK2),
        idx4(dst_p, CHUNK2))
    return _make_dense2()(agg2, deg_col, h2n, W2_l.T, W2_r.T,
                          b2.reshape(1, D))
